# Initial kernel scaffold; baseline (speedup 1.0000x reference)
import jax
import jax.numpy as jnp
from jax.experimental import pallas as pl


def kernel(x, edge_index, batch, W1, b1, W2, b2, W3, b3):
    raise NotImplementedError("write your pallas kernel here")


# speedup vs baseline: 45.3233x; 45.3233x over previous
import functools

import jax
import jax.numpy as jnp
from jax import lax
from jax.experimental import pallas as pl
from jax.experimental.pallas import tpu as pltpu
from jax.experimental.pallas import tpu_sc as plsc

NC = 2
NS = 16
NW = NC * NS
LANES = 16
IPR = 128
RPC = 8
CHUNK = IPR * RPC


def _sc_mesh():
    return plsc.VectorSubcoreMesh(
        core_axis_name="c", subcore_axis_name="s",
        num_cores=NC, num_subcores=NS)


def _agg_partials(y, src3, dst3, zeros2):
    NP, F = y.shape
    R = src3.shape[0]
    K = R // (NW * RPC)

    @functools.partial(
        pl.kernel,
        out_type=jax.ShapeDtypeStruct((NC, NP, F), jnp.float32),
        mesh=_sc_mesh(),
        compiler_params=pltpu.CompilerParams(use_tc_tiling_on_sc=False),
        scratch_types=[
            pltpu.VMEM_SHARED((NP, F), jnp.float32),
            pltpu.VMEM((RPC, IPR), jnp.int32),
            pltpu.VMEM((RPC, IPR), jnp.int32),
            pltpu.VMEM((CHUNK, F), jnp.float32),
            pltpu.SemaphoreType.DMA,
        ],
    )
    def agg_k(y_hbm, src_hbm, dst_hbm, zeros_hbm, out_hbm,
              acc, src_v, dst_v, rows_v, gsem):
        c = lax.axis_index("c")
        s = lax.axis_index("s")
        wid = s * NC + c

        @pl.when(s == 0)
        def _():
            pltpu.sync_copy(zeros_hbm, acc)
        plsc.subcore_barrier()

        def body(k, carry):
            row0 = wid * (K * RPC) + k * RPC
            pltpu.sync_copy(src_hbm.at[pl.ds(row0, RPC)], src_v)
            pltpu.sync_copy(dst_hbm.at[pl.ds(row0, RPC)], dst_v)
            descs = [
                pltpu.async_copy(y_hbm.at[src_v.at[j]],
                                 rows_v.at[pl.ds(j * IPR, IPR)], gsem)
                for j in range(RPC)
            ]
            for d in descs:
                d.wait()
            for j in range(RPC):
                pltpu.sync_copy(rows_v.at[pl.ds(j * IPR, IPR)],
                                acc.at[dst_v.at[j]], add=True)
            return carry

        lax.fori_loop(0, K, body, 0)
        plsc.subcore_barrier()

        @pl.when(s == 0)
        def _():
            pltpu.sync_copy(acc, out_hbm.at[c])

    return agg_k(y, src3, dst3, zeros2)


def _tc_prep(x_pad, d0, d1, W1):
    NP = x_pad.shape[0]
    G = 16
    B = NP // G
    Fi, Fo = W1.shape

    def body(x_ref, d0_ref, d1_ref, w_ref, dinv_ref, y_ref):
        dinv = lax.rsqrt(d0_ref[...] + d1_ref[...] + 1.0)
        dinv_ref[...] = dinv
        y_ref[...] = dinv * jnp.dot(x_ref[...], w_ref[...],
                                    preferred_element_type=jnp.float32)

    return pl.pallas_call(
        body,
        grid=(G,),
        in_specs=[
            pl.BlockSpec((B, Fi), lambda i: (i, 0)),
            pl.BlockSpec((B, 1), lambda i: (i, 0)),
            pl.BlockSpec((B, 1), lambda i: (i, 0)),
            pl.BlockSpec((Fi, Fo), lambda i: (0, 0)),
        ],
        out_specs=[
            pl.BlockSpec((B, 1), lambda i: (i, 0)),
            pl.BlockSpec((B, Fo), lambda i: (i, 0)),
        ],
        out_shape=[
            jax.ShapeDtypeStruct((NP, 1), jnp.float32),
            jax.ShapeDtypeStruct((NP, Fo), jnp.float32),
        ],
    )(x_pad, d0, d1, W1)


def _tc_mid(a0, a1, y, dinv, b2d, W):
    NP, F = y.shape
    G = 16
    B = NP // G
    Fo = W.shape[1]

    def body(a0_ref, a1_ref, y_ref, dinv_ref, b_ref, w_ref, out_ref):
        dinv = dinv_ref[...]
        h = jnp.maximum(
            dinv * (a0_ref[...] + a1_ref[...] + y_ref[...]) + b_ref[...], 0.0)
        out_ref[...] = dinv * jnp.dot(h, w_ref[...],
                                      preferred_element_type=jnp.float32)

    return pl.pallas_call(
        body,
        grid=(G,),
        in_specs=[
            pl.BlockSpec((B, F), lambda i: (i, 0)),
            pl.BlockSpec((B, F), lambda i: (i, 0)),
            pl.BlockSpec((B, F), lambda i: (i, 0)),
            pl.BlockSpec((B, 1), lambda i: (i, 0)),
            pl.BlockSpec((1, F), lambda i: (0, 0)),
            pl.BlockSpec((F, Fo), lambda i: (0, 0)),
        ],
        out_specs=pl.BlockSpec((B, Fo), lambda i: (i, 0)),
        out_shape=jax.ShapeDtypeStruct((NP, Fo), jnp.float32),
    )(a0, a1, y, dinv, b2d, W)


def _tc_fin(a0, a1, y, dinv, b2d, Fo):
    NP, F = y.shape
    G = 16
    B = NP // G

    def body(a0_ref, a1_ref, y_ref, dinv_ref, b_ref, out_ref):
        h = jnp.maximum(
            dinv_ref[...] * (a0_ref[...] + a1_ref[...] + y_ref[...])
            + b_ref[...], 0.0)
        out_ref[...] = h[:, :Fo]

    return pl.pallas_call(
        body,
        grid=(G,),
        in_specs=[
            pl.BlockSpec((B, F), lambda i: (i, 0)),
            pl.BlockSpec((B, F), lambda i: (i, 0)),
            pl.BlockSpec((B, F), lambda i: (i, 0)),
            pl.BlockSpec((B, 1), lambda i: (i, 0)),
            pl.BlockSpec((1, F), lambda i: (0, 0)),
        ],
        out_specs=pl.BlockSpec((B, Fo), lambda i: (i, 0)),
        out_shape=jax.ShapeDtypeStruct((NP, Fo), jnp.float32),
    )(a0, a1, y, dinv, b2d)


def kernel(x, edge_index, batch, W1, b1, W2, b2, W3, b3):
    N = x.shape[0]
    E = edge_index.shape[1]
    F = 16
    NP = ((N + 127) // 128) * 128 + 128

    src = edge_index[0].astype(jnp.int32)
    dst = edge_index[1].astype(jnp.int32)

    per_round = NW * CHUNK
    K = (E + per_round - 1) // per_round
    E_pad = K * per_round
    pad = E_pad - E
    pad_idx = (NP - 128) + (jnp.arange(pad, dtype=jnp.int32) % 128)
    src3 = jnp.concatenate([src, pad_idx]).reshape(-1, IPR)
    dst3 = jnp.concatenate([dst, pad_idx]).reshape(-1, IPR)

    zeros2 = jnp.zeros((NP, F), jnp.float32)
    x_pad = jnp.pad(x, ((0, NP - N), (0, 0)))

    W3p = jnp.pad(W3, ((0, 0), (0, F - W3.shape[1])))
    b1_2d = b1.reshape(1, F)
    b2_2d = b2.reshape(1, F)
    b3_2d = jnp.pad(b3, (0, F - b3.shape[0])).reshape(1, F)

    ones_tab = jnp.ones((NP, F), jnp.float32)
    degp = _agg_partials(ones_tab, src3, dst3, zeros2)
    d0 = degp[0, :, 0].reshape(NP, 1)
    d1 = degp[1, :, 0].reshape(NP, 1)

    dinv, y1 = _tc_prep(x_pad, d0, d1, W1)

    p = _agg_partials(y1, src3, dst3, zeros2)
    y2 = _tc_mid(p[0], p[1], y1, dinv, b1_2d, W2)

    p = _agg_partials(y2, src3, dst3, zeros2)
    y3 = _tc_mid(p[0], p[1], y2, dinv, b2_2d, W3p)

    p = _agg_partials(y3, src3, dst3, zeros2)
    h = _tc_fin(p[0], p[1], y3, dinv, b3_2d, b3.shape[0])

    return h[:N]


# speedup vs baseline: 47.9442x; 1.0578x over previous
import functools

import jax
import jax.numpy as jnp
from jax import lax
from jax.experimental import pallas as pl
from jax.experimental.pallas import tpu as pltpu
from jax.experimental.pallas import tpu_sc as plsc

NC = 2
NS = 16
NW = NC * NS
LANES = 16
IPR = 128
RPC = 4
CHUNK = IPR * RPC


def _sc_mesh():
    return plsc.VectorSubcoreMesh(
        core_axis_name="c", subcore_axis_name="s",
        num_cores=NC, num_subcores=NS)


def _agg_partials(y, src3, dst3, zeros2):
    NP, F = y.shape
    R = src3.shape[0]
    K = R // (NW * RPC)

    @functools.partial(
        pl.kernel,
        out_type=jax.ShapeDtypeStruct((NC, NP, F), jnp.float32),
        mesh=_sc_mesh(),
        compiler_params=pltpu.CompilerParams(use_tc_tiling_on_sc=False),
        scratch_types=[
            pltpu.VMEM_SHARED((NP, F), jnp.float32),
            pltpu.VMEM((2, RPC, IPR), jnp.int32),
            pltpu.VMEM((2, RPC, IPR), jnp.int32),
            pltpu.VMEM((2, CHUNK, F), jnp.float32),
            pltpu.SemaphoreType.DMA,
            pltpu.SemaphoreType.DMA,
            pltpu.SemaphoreType.DMA,
            pltpu.SemaphoreType.DMA,
        ],
    )
    def agg_k(y_hbm, src_hbm, dst_hbm, zeros_hbm, out_hbm,
              acc, src_v, dst_v, rows_v, gsem0, gsem1, ssem0, ssem1):
        c = lax.axis_index("c")
        s = lax.axis_index("s")
        wid = s * NC + c
        gsems = (gsem0, gsem1)
        ssems = (ssem0, ssem1)

        @pl.when(s == 0)
        def _():
            pltpu.sync_copy(zeros_hbm, acc)
        plsc.subcore_barrier()

        def load_and_gather(b, chunk):
            row0 = wid * (K * RPC) + chunk * RPC
            pltpu.sync_copy(src_hbm.at[pl.ds(row0, RPC)], src_v.at[b])
            pltpu.sync_copy(dst_hbm.at[pl.ds(row0, RPC)], dst_v.at[b])
            return [
                pltpu.async_copy(y_hbm.at[src_v.at[b, j]],
                                 rows_v.at[b, pl.ds(j * IPR, IPR)], gsems[b])
                for j in range(RPC)
            ]

        def fire_scatters(b):
            return [
                pltpu.async_copy(rows_v.at[b, pl.ds(j * IPR, IPR)],
                                 acc.at[dst_v.at[b, j]], ssems[b], add=True)
                for j in range(RPC)
            ]

        def body(k, carry):
            g0 = load_and_gather(0, 2 * k)
            g1 = load_and_gather(1, 2 * k + 1)
            for d in g0:
                d.wait()
            s0 = fire_scatters(0)
            for d in g1:
                d.wait()
            s1 = fire_scatters(1)
            for d in s0:
                d.wait()
            for d in s1:
                d.wait()
            return carry

        lax.fori_loop(0, K // 2, body, 0)
        plsc.subcore_barrier()

        @pl.when(s == 0)
        def _():
            pltpu.sync_copy(acc, out_hbm.at[c])

    return agg_k(y, src3, dst3, zeros2)


def _tc_prep(x_pad, d0, d1, W1):
    NP = x_pad.shape[0]
    G = 16
    B = NP // G
    Fi, Fo = W1.shape

    def body(x_ref, d0_ref, d1_ref, w_ref, dinv_ref, y_ref):
        dinv = lax.rsqrt(d0_ref[...] + d1_ref[...] + 1.0)
        dinv_ref[...] = dinv
        y_ref[...] = dinv * jnp.dot(x_ref[...], w_ref[...],
                                    preferred_element_type=jnp.float32)

    return pl.pallas_call(
        body,
        grid=(G,),
        in_specs=[
            pl.BlockSpec((B, Fi), lambda i: (i, 0)),
            pl.BlockSpec((B, 1), lambda i: (i, 0)),
            pl.BlockSpec((B, 1), lambda i: (i, 0)),
            pl.BlockSpec((Fi, Fo), lambda i: (0, 0)),
        ],
        out_specs=[
            pl.BlockSpec((B, 1), lambda i: (i, 0)),
            pl.BlockSpec((B, Fo), lambda i: (i, 0)),
        ],
        out_shape=[
            jax.ShapeDtypeStruct((NP, 1), jnp.float32),
            jax.ShapeDtypeStruct((NP, Fo), jnp.float32),
        ],
    )(x_pad, d0, d1, W1)


def _tc_mid(a0, a1, y, dinv, b2d, W):
    NP, F = y.shape
    G = 16
    B = NP // G
    Fo = W.shape[1]

    def body(a0_ref, a1_ref, y_ref, dinv_ref, b_ref, w_ref, out_ref):
        dinv = dinv_ref[...]
        h = jnp.maximum(
            dinv * (a0_ref[...] + a1_ref[...] + y_ref[...]) + b_ref[...], 0.0)
        out_ref[...] = dinv * jnp.dot(h, w_ref[...],
                                      preferred_element_type=jnp.float32)

    return pl.pallas_call(
        body,
        grid=(G,),
        in_specs=[
            pl.BlockSpec((B, F), lambda i: (i, 0)),
            pl.BlockSpec((B, F), lambda i: (i, 0)),
            pl.BlockSpec((B, F), lambda i: (i, 0)),
            pl.BlockSpec((B, 1), lambda i: (i, 0)),
            pl.BlockSpec((1, F), lambda i: (0, 0)),
            pl.BlockSpec((F, Fo), lambda i: (0, 0)),
        ],
        out_specs=pl.BlockSpec((B, Fo), lambda i: (i, 0)),
        out_shape=jax.ShapeDtypeStruct((NP, Fo), jnp.float32),
    )(a0, a1, y, dinv, b2d, W)


def _tc_fin(a0, a1, y, dinv, b2d, Fo):
    NP, F = y.shape
    G = 16
    B = NP // G

    def body(a0_ref, a1_ref, y_ref, dinv_ref, b_ref, out_ref):
        h = jnp.maximum(
            dinv_ref[...] * (a0_ref[...] + a1_ref[...] + y_ref[...])
            + b_ref[...], 0.0)
        out_ref[...] = h[:, :Fo]

    return pl.pallas_call(
        body,
        grid=(G,),
        in_specs=[
            pl.BlockSpec((B, F), lambda i: (i, 0)),
            pl.BlockSpec((B, F), lambda i: (i, 0)),
            pl.BlockSpec((B, F), lambda i: (i, 0)),
            pl.BlockSpec((B, 1), lambda i: (i, 0)),
            pl.BlockSpec((1, F), lambda i: (0, 0)),
        ],
        out_specs=pl.BlockSpec((B, Fo), lambda i: (i, 0)),
        out_shape=jax.ShapeDtypeStruct((NP, Fo), jnp.float32),
    )(a0, a1, y, dinv, b2d)


def kernel(x, edge_index, batch, W1, b1, W2, b2, W3, b3):
    N = x.shape[0]
    E = edge_index.shape[1]
    F = 16
    NP = ((N + 127) // 128) * 128 + 128

    src = edge_index[0].astype(jnp.int32)
    dst = edge_index[1].astype(jnp.int32)

    per_round = NW * CHUNK
    K = (E + per_round - 1) // per_round
    K = K + (K % 2)
    E_pad = K * per_round
    pad = E_pad - E
    pad_idx = (NP - 128) + (jnp.arange(pad, dtype=jnp.int32) % 128)
    src3 = jnp.concatenate([src, pad_idx]).reshape(-1, IPR)
    dst3 = jnp.concatenate([dst, pad_idx]).reshape(-1, IPR)

    zeros2 = jnp.zeros((NP, F), jnp.float32)
    x_pad = jnp.pad(x, ((0, NP - N), (0, 0)))

    W3p = jnp.pad(W3, ((0, 0), (0, F - W3.shape[1])))
    b1_2d = b1.reshape(1, F)
    b2_2d = b2.reshape(1, F)
    b3_2d = jnp.pad(b3, (0, F - b3.shape[0])).reshape(1, F)

    ones_tab = jnp.ones((NP, F), jnp.float32)
    degp = _agg_partials(ones_tab, src3, dst3, zeros2)
    d0 = degp[0, :, 0].reshape(NP, 1)
    d1 = degp[1, :, 0].reshape(NP, 1)

    dinv, y1 = _tc_prep(x_pad, d0, d1, W1)

    p = _agg_partials(y1, src3, dst3, zeros2)
    y2 = _tc_mid(p[0], p[1], y1, dinv, b1_2d, W2)

    p = _agg_partials(y2, src3, dst3, zeros2)
    y3 = _tc_mid(p[0], p[1], y2, dinv, b2_2d, W3p)

    p = _agg_partials(y3, src3, dst3, zeros2)
    h = _tc_fin(p[0], p[1], y3, dinv, b3_2d, b3.shape[0])

    return h[:N]


# speedup vs baseline: 62.7437x; 1.3087x over previous
import functools

import jax
import jax.numpy as jnp
from jax import lax
from jax.experimental import pallas as pl
from jax.experimental.pallas import tpu as pltpu
from jax.experimental.pallas import tpu_sc as plsc

NC = 2
NS = 16
NW = NC * NS
LANES = 16
IPR = 128
RPC = 4
CHUNK = IPR * RPC


def _sc_mesh():
    return plsc.VectorSubcoreMesh(
        core_axis_name="c", subcore_axis_name="s",
        num_cores=NC, num_subcores=NS)


def _agg_partials(y, edges3, zeros2):
    NP, F = y.shape
    R = edges3.shape[0]
    K = R // (NW * RPC)

    @functools.partial(
        pl.kernel,
        out_type=jax.ShapeDtypeStruct((NC, NP, F), jnp.float32),
        mesh=_sc_mesh(),
        compiler_params=pltpu.CompilerParams(use_tc_tiling_on_sc=False),
        scratch_types=[
            pltpu.VMEM_SHARED((NP, F), jnp.float32),
            pltpu.VMEM((2, RPC, 2, IPR), jnp.int32),
            pltpu.VMEM((2, CHUNK, F), jnp.float32),
            pltpu.SemaphoreType.DMA,
            pltpu.SemaphoreType.DMA,
            pltpu.SemaphoreType.DMA,
            pltpu.SemaphoreType.DMA,
        ],
    )
    def agg_k(y_hbm, edges_hbm, zeros_hbm, out_hbm,
              acc, idx_v, rows_v, gsem0, gsem1, ssem0, ssem1):
        c = lax.axis_index("c")
        s = lax.axis_index("s")
        wid = s * NC + c
        gsems = (gsem0, gsem1)
        ssems = (ssem0, ssem1)

        @pl.when(s == 0)
        def _():
            pltpu.sync_copy(zeros_hbm, acc)
        plsc.subcore_barrier()

        def load_and_gather(b, chunk):
            row0 = wid * (K * RPC) + chunk * RPC
            pltpu.sync_copy(edges_hbm.at[pl.ds(row0, RPC)], idx_v.at[b])
            return [
                pltpu.async_copy(y_hbm.at[idx_v.at[b, j, 0]],
                                 rows_v.at[b, pl.ds(j * IPR, IPR)], gsems[b])
                for j in range(RPC)
            ]

        def fire_scatters(b):
            return [
                pltpu.async_copy(rows_v.at[b, pl.ds(j * IPR, IPR)],
                                 acc.at[idx_v.at[b, j, 1]], ssems[b], add=True)
                for j in range(RPC)
            ]

        def body(k, carry):
            g0 = load_and_gather(0, 2 * k)
            g1 = load_and_gather(1, 2 * k + 1)
            for d in g0:
                d.wait()
            s0 = fire_scatters(0)
            for d in g1:
                d.wait()
            s1 = fire_scatters(1)
            for d in s0:
                d.wait()
            for d in s1:
                d.wait()
            return carry

        lax.fori_loop(0, K // 2, body, 0)
        plsc.subcore_barrier()

        @pl.when(s == 0)
        def _():
            pltpu.sync_copy(acc, out_hbm.at[c])

    return agg_k(y, edges3, zeros2)


def _deg_partials(edges3, zeros2, ones2):
    R = edges3.shape[0]
    NP, F = zeros2.shape
    K = R // (NW * RPC)

    @functools.partial(
        pl.kernel,
        out_type=jax.ShapeDtypeStruct((NC, NP, F), jnp.float32),
        mesh=_sc_mesh(),
        compiler_params=pltpu.CompilerParams(use_tc_tiling_on_sc=False),
        scratch_types=[
            pltpu.VMEM_SHARED((NP, F), jnp.float32),
            pltpu.VMEM((2, RPC, 2, IPR), jnp.int32),
            pltpu.VMEM((IPR, F), jnp.float32),
            pltpu.SemaphoreType.DMA,
            pltpu.SemaphoreType.DMA,
        ],
    )
    def deg_k(edges_hbm, zeros_hbm, ones_hbm, out_hbm,
              acc, idx_v, ones_v, ssem0, ssem1):
        c = lax.axis_index("c")
        s = lax.axis_index("s")
        wid = s * NC + c
        ssems = (ssem0, ssem1)

        @pl.when(s == 0)
        def _():
            pltpu.sync_copy(zeros_hbm, acc)
        pltpu.sync_copy(ones_hbm, ones_v)
        plsc.subcore_barrier()

        def load_idx(b, chunk):
            row0 = wid * (K * RPC) + chunk * RPC
            pltpu.sync_copy(edges_hbm.at[pl.ds(row0, RPC)], idx_v.at[b])

        def fire_scatters(b):
            return [
                pltpu.async_copy(ones_v, acc.at[idx_v.at[b, j, 1]],
                                 ssems[b], add=True)
                for j in range(RPC)
            ]

        def body(k, carry):
            load_idx(0, 2 * k)
            s0 = fire_scatters(0)
            load_idx(1, 2 * k + 1)
            s1 = fire_scatters(1)
            for d in s0:
                d.wait()
            for d in s1:
                d.wait()
            return carry

        lax.fori_loop(0, K // 2, body, 0)
        plsc.subcore_barrier()

        @pl.when(s == 0)
        def _():
            pltpu.sync_copy(acc, out_hbm.at[c])

    return deg_k(edges3, zeros2, ones2)


def _tc_prep(x_pad, d0, d1, W1):
    NP = x_pad.shape[0]
    G = 16
    B = NP // G
    Fi, Fo = W1.shape

    def body(x_ref, d0_ref, d1_ref, w_ref, dinv_ref, y_ref):
        dinv = lax.rsqrt(d0_ref[...] + d1_ref[...] + 1.0)
        dinv_ref[...] = dinv
        y_ref[...] = dinv * jnp.dot(x_ref[...], w_ref[...],
                                    preferred_element_type=jnp.float32)

    return pl.pallas_call(
        body,
        grid=(G,),
        in_specs=[
            pl.BlockSpec((B, Fi), lambda i: (i, 0)),
            pl.BlockSpec((B, 1), lambda i: (i, 0)),
            pl.BlockSpec((B, 1), lambda i: (i, 0)),
            pl.BlockSpec((Fi, Fo), lambda i: (0, 0)),
        ],
        out_specs=[
            pl.BlockSpec((B, 1), lambda i: (i, 0)),
            pl.BlockSpec((B, Fo), lambda i: (i, 0)),
        ],
        out_shape=[
            jax.ShapeDtypeStruct((NP, 1), jnp.float32),
            jax.ShapeDtypeStruct((NP, Fo), jnp.float32),
        ],
    )(x_pad, d0, d1, W1)


def _tc_mid(a0, a1, y, dinv, b2d, W):
    NP, F = y.shape
    G = 16
    B = NP // G
    Fo = W.shape[1]

    def body(a0_ref, a1_ref, y_ref, dinv_ref, b_ref, w_ref, out_ref):
        dinv = dinv_ref[...]
        h = jnp.maximum(
            dinv * (a0_ref[...] + a1_ref[...] + y_ref[...]) + b_ref[...], 0.0)
        out_ref[...] = dinv * jnp.dot(h, w_ref[...],
                                      preferred_element_type=jnp.float32)

    return pl.pallas_call(
        body,
        grid=(G,),
        in_specs=[
            pl.BlockSpec((B, F), lambda i: (i, 0)),
            pl.BlockSpec((B, F), lambda i: (i, 0)),
            pl.BlockSpec((B, F), lambda i: (i, 0)),
            pl.BlockSpec((B, 1), lambda i: (i, 0)),
            pl.BlockSpec((1, F), lambda i: (0, 0)),
            pl.BlockSpec((F, Fo), lambda i: (0, 0)),
        ],
        out_specs=pl.BlockSpec((B, Fo), lambda i: (i, 0)),
        out_shape=jax.ShapeDtypeStruct((NP, Fo), jnp.float32),
    )(a0, a1, y, dinv, b2d, W)


def _tc_fin(a0, a1, y, dinv, b2d, Fo):
    NP, F = y.shape
    G = 16
    B = NP // G

    def body(a0_ref, a1_ref, y_ref, dinv_ref, b_ref, out_ref):
        h = jnp.maximum(
            dinv_ref[...] * (a0_ref[...] + a1_ref[...] + y_ref[...])
            + b_ref[...], 0.0)
        out_ref[...] = h[:, :Fo]

    return pl.pallas_call(
        body,
        grid=(G,),
        in_specs=[
            pl.BlockSpec((B, F), lambda i: (i, 0)),
            pl.BlockSpec((B, F), lambda i: (i, 0)),
            pl.BlockSpec((B, F), lambda i: (i, 0)),
            pl.BlockSpec((B, 1), lambda i: (i, 0)),
            pl.BlockSpec((1, F), lambda i: (0, 0)),
        ],
        out_specs=pl.BlockSpec((B, Fo), lambda i: (i, 0)),
        out_shape=jax.ShapeDtypeStruct((NP, Fo), jnp.float32),
    )(a0, a1, y, dinv, b2d)


def kernel(x, edge_index, batch, W1, b1, W2, b2, W3, b3):
    N = x.shape[0]
    E = edge_index.shape[1]
    F = 16
    NP = ((N + 127) // 128) * 128 + 128

    src = edge_index[0].astype(jnp.int32)
    dst = edge_index[1].astype(jnp.int32)

    per_round = NW * CHUNK
    K = (E + per_round - 1) // per_round
    K = K + (K % 2)
    E_pad = K * per_round
    pad = E_pad - E
    pad_idx = (NP - 128) + (jnp.arange(pad, dtype=jnp.int32) % 128)
    src3 = jnp.concatenate([src, pad_idx]).reshape(-1, IPR)
    dst3 = jnp.concatenate([dst, pad_idx]).reshape(-1, IPR)
    edges3 = jnp.stack([src3, dst3], axis=1)

    zeros2 = jnp.zeros((NP, F), jnp.float32)
    x_pad = jnp.pad(x, ((0, NP - N), (0, 0)))

    W3p = jnp.pad(W3, ((0, 0), (0, F - W3.shape[1])))
    b1_2d = b1.reshape(1, F)
    b2_2d = b2.reshape(1, F)
    b3_2d = jnp.pad(b3, (0, F - b3.shape[0])).reshape(1, F)

    ones2 = jnp.ones((IPR, F), jnp.float32)
    degp = _deg_partials(edges3, zeros2, ones2)
    d0 = degp[0, :, 0].reshape(NP, 1)
    d1 = degp[1, :, 0].reshape(NP, 1)

    dinv, y1 = _tc_prep(x_pad, d0, d1, W1)

    p = _agg_partials(y1, edges3, zeros2)
    y2 = _tc_mid(p[0], p[1], y1, dinv, b1_2d, W2)

    p = _agg_partials(y2, edges3, zeros2)
    y3 = _tc_mid(p[0], p[1], y2, dinv, b2_2d, W3p)

    p = _agg_partials(y3, edges3, zeros2)
    h = _tc_fin(p[0], p[1], y3, dinv, b3_2d, b3.shape[0])

    return h[:N]


# speedup vs baseline: 77.7840x; 1.2397x over previous
import functools

import jax
import jax.numpy as jnp
from jax import lax
from jax.experimental import pallas as pl
from jax.experimental.pallas import tpu as pltpu
from jax.experimental.pallas import tpu_sc as plsc

NC = 2
NS = 16
NW = NC * NS
LANES = 16
IPR = 128
RPC = 6
CHUNK = IPR * RPC


def _sc_mesh():
    return plsc.VectorSubcoreMesh(
        core_axis_name="c", subcore_axis_name="s",
        num_cores=NC, num_subcores=NS)


def _agg_partials(y, edges3, zeros2):
    NP, F = y.shape
    R = edges3.shape[0]
    K = R // (NW * RPC)

    @functools.partial(
        pl.kernel,
        out_type=jax.ShapeDtypeStruct((NC, NP, F), jnp.float32),
        mesh=_sc_mesh(),
        compiler_params=pltpu.CompilerParams(use_tc_tiling_on_sc=False),
        scratch_types=[
            pltpu.VMEM_SHARED((NP, F), jnp.float32),
            pltpu.VMEM((2 * RPC, 2, IPR), jnp.int32),
            pltpu.VMEM((2 * RPC, 2, IPR), jnp.int32),
            pltpu.VMEM((2, CHUNK, F), jnp.float32),
            pltpu.SemaphoreType.DMA,
            pltpu.SemaphoreType.DMA,
            pltpu.SemaphoreType.DMA,
            pltpu.SemaphoreType.DMA,
            pltpu.SemaphoreType.DMA,
            pltpu.SemaphoreType.DMA,
        ],
    )
    def agg_k(y_hbm, edges_hbm, zeros_hbm, out_hbm,
              acc, p0_v, p1_v, rows_v,
              isem0, isem1, gsem0, gsem1, ssem0, ssem1):
        c = lax.axis_index("c")
        s = lax.axis_index("s")
        wid = s * NC + c
        gsems = (gsem0, gsem1)
        ssems = (ssem0, ssem1)
        wbase = wid * (K * RPC)

        @pl.when(s == 0)
        def _():
            pltpu.sync_copy(zeros_hbm, acc)
        plsc.subcore_barrier()

        def fire_idx_pair(pref, isem, chunk):
            row0 = wbase + chunk * RPC
            pltpu.async_copy(edges_hbm.at[pl.ds(row0, 2 * RPC)], pref, isem)

        def drain_idx(pref, isem):
            pltpu.make_async_copy(edges_hbm.at[pl.ds(0, 2 * RPC)],
                                  pref, isem).wait()

        def fire_gathers(rb, pref, half):
            return [
                pltpu.async_copy(y_hbm.at[pref.at[half * RPC + j, 0]],
                                 rows_v.at[rb, pl.ds(j * IPR, IPR)], gsems[rb])
                for j in range(RPC)
            ]

        def fire_scatters(rb, pref, half):
            return [
                pltpu.async_copy(rows_v.at[rb, pl.ds(j * IPR, IPR)],
                                 acc.at[pref.at[half * RPC + j, 1]],
                                 ssems[rb], add=True)
                for j in range(RPC)
            ]

        fire_idx_pair(p0_v, isem0, 0)

        def body(k, carry):
            c0 = 4 * k
            drain_idx(p0_v, isem0)
            fire_idx_pair(p1_v, isem1, jnp.minimum(c0 + 2, K - 2))
            g0 = fire_gathers(0, p0_v, 0)
            g1 = fire_gathers(1, p0_v, 1)
            for d in g0:
                d.wait()
            s0 = fire_scatters(0, p0_v, 0)
            for d in g1:
                d.wait()
            s1 = fire_scatters(1, p0_v, 1)
            drain_idx(p1_v, isem1)
            for d in s0:
                d.wait()
            g2 = fire_gathers(0, p1_v, 0)
            for d in s1:
                d.wait()
            g3 = fire_gathers(1, p1_v, 1)
            fire_idx_pair(p0_v, isem0, jnp.minimum(c0 + 4, K - 2))
            for d in g2:
                d.wait()
            s2 = fire_scatters(0, p1_v, 0)
            for d in g3:
                d.wait()
            s3 = fire_scatters(1, p1_v, 1)
            for d in s2:
                d.wait()
            for d in s3:
                d.wait()
            return carry

        lax.fori_loop(0, K // 4, body, 0)
        drain_idx(p0_v, isem0)
        plsc.subcore_barrier()

        @pl.when(s == 0)
        def _():
            pltpu.sync_copy(acc, out_hbm.at[c])

    return agg_k(y, edges3, zeros2)


def _deg_partials(edges3, zeros2, ones2):
    R = edges3.shape[0]
    NP, F = zeros2.shape
    K = R // (NW * RPC)

    @functools.partial(
        pl.kernel,
        out_type=jax.ShapeDtypeStruct((NC, NP, F), jnp.float32),
        mesh=_sc_mesh(),
        compiler_params=pltpu.CompilerParams(use_tc_tiling_on_sc=False),
        scratch_types=[
            pltpu.VMEM_SHARED((NP, F), jnp.float32),
            pltpu.VMEM((2, RPC, 2, IPR), jnp.int32),
            pltpu.VMEM((IPR, F), jnp.float32),
            pltpu.SemaphoreType.DMA,
            pltpu.SemaphoreType.DMA,
        ],
    )
    def deg_k(edges_hbm, zeros_hbm, ones_hbm, out_hbm,
              acc, idx_v, ones_v, ssem0, ssem1):
        c = lax.axis_index("c")
        s = lax.axis_index("s")
        wid = s * NC + c
        ssems = (ssem0, ssem1)

        @pl.when(s == 0)
        def _():
            pltpu.sync_copy(zeros_hbm, acc)
        pltpu.sync_copy(ones_hbm, ones_v)
        plsc.subcore_barrier()

        def load_idx(b, chunk):
            row0 = wid * (K * RPC) + chunk * RPC
            pltpu.sync_copy(edges_hbm.at[pl.ds(row0, RPC)], idx_v.at[b])

        def fire_scatters(b):
            return [
                pltpu.async_copy(ones_v, acc.at[idx_v.at[b, j, 1]],
                                 ssems[b], add=True)
                for j in range(RPC)
            ]

        def body(k, carry):
            load_idx(0, 2 * k)
            s0 = fire_scatters(0)
            load_idx(1, 2 * k + 1)
            s1 = fire_scatters(1)
            for d in s0:
                d.wait()
            for d in s1:
                d.wait()
            return carry

        lax.fori_loop(0, K // 2, body, 0)
        plsc.subcore_barrier()

        @pl.when(s == 0)
        def _():
            pltpu.sync_copy(acc, out_hbm.at[c])

    return deg_k(edges3, zeros2, ones2)


def _tc_prep(x_pad, d0, d1, W1):
    NP = x_pad.shape[0]
    G = 16
    B = NP // G
    Fi, Fo = W1.shape

    def body(x_ref, d0_ref, d1_ref, w_ref, dinv_ref, y_ref):
        dinv = lax.rsqrt(d0_ref[...] + d1_ref[...] + 1.0)
        dinv_ref[...] = dinv
        y_ref[...] = dinv * jnp.dot(x_ref[...], w_ref[...],
                                    preferred_element_type=jnp.float32)

    return pl.pallas_call(
        body,
        grid=(G,),
        in_specs=[
            pl.BlockSpec((B, Fi), lambda i: (i, 0)),
            pl.BlockSpec((B, 1), lambda i: (i, 0)),
            pl.BlockSpec((B, 1), lambda i: (i, 0)),
            pl.BlockSpec((Fi, Fo), lambda i: (0, 0)),
        ],
        out_specs=[
            pl.BlockSpec((B, 1), lambda i: (i, 0)),
            pl.BlockSpec((B, Fo), lambda i: (i, 0)),
        ],
        out_shape=[
            jax.ShapeDtypeStruct((NP, 1), jnp.float32),
            jax.ShapeDtypeStruct((NP, Fo), jnp.float32),
        ],
    )(x_pad, d0, d1, W1)


def _tc_mid(a0, a1, y, dinv, b2d, W):
    NP, F = y.shape
    G = 16
    B = NP // G
    Fo = W.shape[1]

    def body(a0_ref, a1_ref, y_ref, dinv_ref, b_ref, w_ref, out_ref):
        dinv = dinv_ref[...]
        h = jnp.maximum(
            dinv * (a0_ref[...] + a1_ref[...] + y_ref[...]) + b_ref[...], 0.0)
        out_ref[...] = dinv * jnp.dot(h, w_ref[...],
                                      preferred_element_type=jnp.float32)

    return pl.pallas_call(
        body,
        grid=(G,),
        in_specs=[
            pl.BlockSpec((B, F), lambda i: (i, 0)),
            pl.BlockSpec((B, F), lambda i: (i, 0)),
            pl.BlockSpec((B, F), lambda i: (i, 0)),
            pl.BlockSpec((B, 1), lambda i: (i, 0)),
            pl.BlockSpec((1, F), lambda i: (0, 0)),
            pl.BlockSpec((F, Fo), lambda i: (0, 0)),
        ],
        out_specs=pl.BlockSpec((B, Fo), lambda i: (i, 0)),
        out_shape=jax.ShapeDtypeStruct((NP, Fo), jnp.float32),
    )(a0, a1, y, dinv, b2d, W)


def _tc_fin(a0, a1, y, dinv, b2d, Fo):
    NP, F = y.shape
    G = 16
    B = NP // G

    def body(a0_ref, a1_ref, y_ref, dinv_ref, b_ref, out_ref):
        h = jnp.maximum(
            dinv_ref[...] * (a0_ref[...] + a1_ref[...] + y_ref[...])
            + b_ref[...], 0.0)
        out_ref[...] = h[:, :Fo]

    return pl.pallas_call(
        body,
        grid=(G,),
        in_specs=[
            pl.BlockSpec((B, F), lambda i: (i, 0)),
            pl.BlockSpec((B, F), lambda i: (i, 0)),
            pl.BlockSpec((B, F), lambda i: (i, 0)),
            pl.BlockSpec((B, 1), lambda i: (i, 0)),
            pl.BlockSpec((1, F), lambda i: (0, 0)),
        ],
        out_specs=pl.BlockSpec((B, Fo), lambda i: (i, 0)),
        out_shape=jax.ShapeDtypeStruct((NP, Fo), jnp.float32),
    )(a0, a1, y, dinv, b2d)


def kernel(x, edge_index, batch, W1, b1, W2, b2, W3, b3):
    N = x.shape[0]
    E = edge_index.shape[1]
    F = 16
    NP = ((N + 127) // 128) * 128 + 128

    src = edge_index[0].astype(jnp.int32)
    dst = edge_index[1].astype(jnp.int32)

    per_round = NW * CHUNK
    K = (E + per_round - 1) // per_round
    K = ((K + 3) // 4) * 4
    E_pad = K * per_round
    pad = E_pad - E
    pad_idx = (NP - 128) + (jnp.arange(pad, dtype=jnp.int32) % 128)
    src3 = jnp.concatenate([src, pad_idx]).reshape(-1, IPR)
    dst3 = jnp.concatenate([dst, pad_idx]).reshape(-1, IPR)
    edges3 = jnp.stack([src3, dst3], axis=1)

    zeros2 = jnp.zeros((NP, F), jnp.float32)
    x_pad = jnp.pad(x, ((0, NP - N), (0, 0)))

    W3p = jnp.pad(W3, ((0, 0), (0, F - W3.shape[1])))
    b1_2d = b1.reshape(1, F)
    b2_2d = b2.reshape(1, F)
    b3_2d = jnp.pad(b3, (0, F - b3.shape[0])).reshape(1, F)

    ones2 = jnp.ones((IPR, F), jnp.float32)
    degp = _deg_partials(edges3, zeros2, ones2)
    d0 = degp[0, :, 0].reshape(NP, 1)
    d1 = degp[1, :, 0].reshape(NP, 1)

    dinv, y1 = _tc_prep(x_pad, d0, d1, W1)

    p = _agg_partials(y1, edges3, zeros2)
    y2 = _tc_mid(p[0], p[1], y1, dinv, b1_2d, W2)

    p = _agg_partials(y2, edges3, zeros2)
    y3 = _tc_mid(p[0], p[1], y2, dinv, b2_2d, W3p)

    p = _agg_partials(y3, edges3, zeros2)
    h = _tc_fin(p[0], p[1], y3, dinv, b3_2d, b3.shape[0])

    return h[:N]


# speedup vs baseline: 80.0921x; 1.0297x over previous
import functools

import jax
import jax.numpy as jnp
from jax import lax
from jax.experimental import pallas as pl
from jax.experimental.pallas import tpu as pltpu
from jax.experimental.pallas import tpu_sc as plsc

NC = 2
NS = 16
NW = NC * NS
LANES = 16
IPR = 128
RPC = 6
CHUNK = IPR * RPC


def _sc_mesh():
    return plsc.VectorSubcoreMesh(
        core_axis_name="c", subcore_axis_name="s",
        num_cores=NC, num_subcores=NS)


def _agg_partials(y, edges3, zeros2):
    NP, F = y.shape
    R = edges3.shape[0]
    K = R // (NW * RPC)

    @functools.partial(
        pl.kernel,
        out_type=jax.ShapeDtypeStruct((NC, NP, F), jnp.float32),
        mesh=_sc_mesh(),
        compiler_params=pltpu.CompilerParams(use_tc_tiling_on_sc=False),
        scratch_types=[
            pltpu.VMEM_SHARED((NP, F), jnp.float32),
            pltpu.VMEM((2 * RPC, 2, IPR), jnp.int32),
            pltpu.VMEM((2 * RPC, 2, IPR), jnp.int32),
            pltpu.VMEM((2, CHUNK, F), jnp.float32),
            pltpu.SemaphoreType.DMA,
            pltpu.SemaphoreType.DMA,
            pltpu.SemaphoreType.DMA,
            pltpu.SemaphoreType.DMA,
            pltpu.SemaphoreType.DMA,
            pltpu.SemaphoreType.DMA,
        ],
    )
    def agg_k(y_hbm, edges_hbm, zeros_hbm, out_hbm,
              acc, p0_v, p1_v, rows_v,
              isem0, isem1, gsem0, gsem1, ssem0, ssem1):
        c = lax.axis_index("c")
        s = lax.axis_index("s")
        wid = s * NC + c
        gsems = (gsem0, gsem1)
        ssems = (ssem0, ssem1)
        wbase = wid * (K * RPC)

        @pl.when(s == 0)
        def _():
            pltpu.sync_copy(zeros_hbm, acc)
        plsc.subcore_barrier()

        def fire_idx_pair(pref, isem, chunk):
            row0 = wbase + chunk * RPC
            pltpu.async_copy(edges_hbm.at[pl.ds(row0, 2 * RPC)], pref, isem)

        def drain_idx(pref, isem):
            pltpu.make_async_copy(edges_hbm.at[pl.ds(0, 2 * RPC)],
                                  pref, isem).wait()

        def fire_gathers(rb, pref, half):
            return [
                pltpu.async_copy(y_hbm.at[pref.at[half * RPC + j, 0]],
                                 rows_v.at[rb, pl.ds(j * IPR, IPR)], gsems[rb])
                for j in range(RPC)
            ]

        def fire_scatters(rb, pref, half):
            return [
                pltpu.async_copy(rows_v.at[rb, pl.ds(j * IPR, IPR)],
                                 acc.at[pref.at[half * RPC + j, 1]],
                                 ssems[rb], add=True)
                for j in range(RPC)
            ]

        fire_idx_pair(p0_v, isem0, 0)

        def body(k, carry):
            c0 = 4 * k
            drain_idx(p0_v, isem0)
            fire_idx_pair(p1_v, isem1, jnp.minimum(c0 + 2, K - 2))
            g0 = fire_gathers(0, p0_v, 0)
            g1 = fire_gathers(1, p0_v, 1)
            for d in g0:
                d.wait()
            s0 = fire_scatters(0, p0_v, 0)
            for d in g1:
                d.wait()
            s1 = fire_scatters(1, p0_v, 1)
            drain_idx(p1_v, isem1)
            for d in s0:
                d.wait()
            g2 = fire_gathers(0, p1_v, 0)
            for d in s1:
                d.wait()
            g3 = fire_gathers(1, p1_v, 1)
            fire_idx_pair(p0_v, isem0, jnp.minimum(c0 + 4, K - 2))
            for d in g2:
                d.wait()
            s2 = fire_scatters(0, p1_v, 0)
            for d in g3:
                d.wait()
            s3 = fire_scatters(1, p1_v, 1)
            for d in s2:
                d.wait()
            for d in s3:
                d.wait()
            return carry

        lax.fori_loop(0, K // 4, body, 0)
        drain_idx(p0_v, isem0)
        plsc.subcore_barrier()

        @pl.when(s == 0)
        def _():
            pltpu.sync_copy(acc, out_hbm.at[c])

    return agg_k(y, edges3, zeros2)


def _deg_partials(edges3, zeros2, ones2):
    R = edges3.shape[0]
    NP, F = zeros2.shape
    K = R // (NW * RPC)

    @functools.partial(
        pl.kernel,
        out_type=jax.ShapeDtypeStruct((NC, NP, F), jnp.float32),
        mesh=_sc_mesh(),
        compiler_params=pltpu.CompilerParams(use_tc_tiling_on_sc=False),
        scratch_types=[
            pltpu.VMEM_SHARED((NP, F), jnp.float32),
            pltpu.VMEM((2, RPC, 2, IPR), jnp.int32),
            pltpu.VMEM((IPR, F), jnp.float32),
            pltpu.SemaphoreType.DMA,
            pltpu.SemaphoreType.DMA,
        ],
    )
    def deg_k(edges_hbm, zeros_hbm, ones_hbm, out_hbm,
              acc, idx_v, ones_v, ssem0, ssem1):
        c = lax.axis_index("c")
        s = lax.axis_index("s")
        wid = s * NC + c
        ssems = (ssem0, ssem1)

        @pl.when(s == 0)
        def _():
            pltpu.sync_copy(zeros_hbm, acc)
        pltpu.sync_copy(ones_hbm, ones_v)
        plsc.subcore_barrier()

        def load_idx(b, chunk):
            row0 = wid * (K * RPC) + chunk * RPC
            pltpu.sync_copy(edges_hbm.at[pl.ds(row0, RPC)], idx_v.at[b])

        def fire_scatters(b):
            return [
                pltpu.async_copy(ones_v, acc.at[idx_v.at[b, j, 1]],
                                 ssems[b], add=True)
                for j in range(RPC)
            ]

        def body(k, carry):
            load_idx(0, 2 * k)
            s0 = fire_scatters(0)
            load_idx(1, 2 * k + 1)
            s1 = fire_scatters(1)
            for d in s0:
                d.wait()
            for d in s1:
                d.wait()
            return carry

        lax.fori_loop(0, K // 2, body, 0)
        plsc.subcore_barrier()

        @pl.when(s == 0)
        def _():
            pltpu.sync_copy(acc, out_hbm.at[c])

    return deg_k(edges3, zeros2, ones2)


def _tc_prep(x_pad, d0, d1, W1):
    NP = x_pad.shape[0]
    G = 16
    B = NP // G
    Fi, Fo = W1.shape

    def body(x_ref, d0_ref, d1_ref, w_ref, dinv_ref, y_ref):
        dinv = lax.rsqrt(d0_ref[...] + d1_ref[...] + 1.0)
        dinv_ref[...] = dinv
        y_ref[...] = dinv * jnp.dot(x_ref[...], w_ref[...],
                                    preferred_element_type=jnp.float32)

    return pl.pallas_call(
        body,
        grid=(G,),
        in_specs=[
            pl.BlockSpec((B, Fi), lambda i: (i, 0)),
            pl.BlockSpec((B, 1), lambda i: (i, 0)),
            pl.BlockSpec((B, 1), lambda i: (i, 0)),
            pl.BlockSpec((Fi, Fo), lambda i: (0, 0)),
        ],
        out_specs=[
            pl.BlockSpec((B, 1), lambda i: (i, 0)),
            pl.BlockSpec((B, Fo), lambda i: (i, 0)),
        ],
        out_shape=[
            jax.ShapeDtypeStruct((NP, 1), jnp.float32),
            jax.ShapeDtypeStruct((NP, Fo), jnp.float32),
        ],
    )(x_pad, d0, d1, W1)


def _tc_mid(a0, a1, y, dinv, b2d, W):
    NP, F = y.shape
    G = 16
    B = NP // G
    Fo = W.shape[1]

    def body(a0_ref, a1_ref, y_ref, dinv_ref, b_ref, w_ref, out_ref):
        dinv = dinv_ref[...]
        h = jnp.maximum(
            dinv * (a0_ref[...] + a1_ref[...] + y_ref[...]) + b_ref[...], 0.0)
        out_ref[...] = dinv * jnp.dot(h, w_ref[...],
                                      preferred_element_type=jnp.float32)

    return pl.pallas_call(
        body,
        grid=(G,),
        in_specs=[
            pl.BlockSpec((B, F), lambda i: (i, 0)),
            pl.BlockSpec((B, F), lambda i: (i, 0)),
            pl.BlockSpec((B, F), lambda i: (i, 0)),
            pl.BlockSpec((B, 1), lambda i: (i, 0)),
            pl.BlockSpec((1, F), lambda i: (0, 0)),
            pl.BlockSpec((F, Fo), lambda i: (0, 0)),
        ],
        out_specs=pl.BlockSpec((B, Fo), lambda i: (i, 0)),
        out_shape=jax.ShapeDtypeStruct((NP, Fo), jnp.float32),
    )(a0, a1, y, dinv, b2d, W)


def _tc_fin(a0, a1, y, dinv, b2d, Fo):
    NP, F = y.shape
    G = 16
    B = NP // G

    def body(a0_ref, a1_ref, y_ref, dinv_ref, b_ref, out_ref):
        h = jnp.maximum(
            dinv_ref[...] * (a0_ref[...] + a1_ref[...] + y_ref[...])
            + b_ref[...], 0.0)
        out_ref[...] = h[:, :Fo]

    return pl.pallas_call(
        body,
        grid=(G,),
        in_specs=[
            pl.BlockSpec((B, F), lambda i: (i, 0)),
            pl.BlockSpec((B, F), lambda i: (i, 0)),
            pl.BlockSpec((B, F), lambda i: (i, 0)),
            pl.BlockSpec((B, 1), lambda i: (i, 0)),
            pl.BlockSpec((1, F), lambda i: (0, 0)),
        ],
        out_specs=pl.BlockSpec((B, Fo), lambda i: (i, 0)),
        out_shape=jax.ShapeDtypeStruct((NP, Fo), jnp.float32),
    )(a0, a1, y, dinv, b2d)


def kernel(x, edge_index, batch, W1, b1, W2, b2, W3, b3):
    N = x.shape[0]
    E = edge_index.shape[1]
    F = 16
    NP = ((N + 127) // 128) * 128 + 128

    src = edge_index[0].astype(jnp.int32)
    dst = edge_index[1].astype(jnp.int32)

    per_round = NW * CHUNK
    K = (E + per_round - 1) // per_round
    K = ((K + 3) // 4) * 4
    E_pad = K * per_round
    pad = E_pad - E
    pad_idx = (NP - 128) + (jnp.arange(pad, dtype=jnp.int32) % 128)
    src3 = jnp.concatenate([src, pad_idx]).reshape(-1, IPR)
    dst3 = jnp.concatenate([dst, pad_idx]).reshape(-1, IPR)
    edges3 = jnp.stack([src3, dst3], axis=1)

    zeros2 = jnp.zeros((NP, F), jnp.float32)
    zeros8 = jnp.zeros((NP, 8), jnp.float32)
    x_pad = jnp.pad(x, ((0, NP - N), (0, 0)))

    b1_2d = b1.reshape(1, F)
    b2_2d = b2.reshape(1, F)
    b3_2d = b3.reshape(1, 8)

    ones8 = jnp.ones((IPR, 8), jnp.float32)
    degp = _deg_partials(edges3, zeros8, ones8)
    d0 = degp[0, :, 0].reshape(NP, 1)
    d1 = degp[1, :, 0].reshape(NP, 1)

    dinv, y1 = _tc_prep(x_pad, d0, d1, W1)

    p = _agg_partials(y1, edges3, zeros2)
    y2 = _tc_mid(p[0], p[1], y1, dinv, b1_2d, W2)

    p = _agg_partials(y2, edges3, zeros2)
    y3 = _tc_mid(p[0], p[1], y2, dinv, b2_2d, W3)

    p = _agg_partials(y3, edges3, zeros8)
    h = _tc_fin(p[0], p[1], y3, dinv, b3_2d, b3.shape[0])

    return h[:N]
